# Initial kernel scaffold; baseline (speedup 1.0000x reference)
import jax
import jax.numpy as jnp
from jax.experimental import pallas as pl


def kernel(features, adj, W_fub, b_fub, W_g, b_g):
    raise NotImplementedError("write your pallas kernel here")


# speedup vs baseline: 1.4948x; 1.4948x over previous
import jax
import jax.numpy as jnp
from jax.experimental import pallas as pl
from jax.experimental.pallas import tpu as pltpu

_B, _N, _C = 8, 1024, 512


def _gcn_fpn_body(x_ref, adj_ref, wf_ref, bf_ref, wg1_ref, wg2_ref, bg_ref,
                  o_ref, a_ref):
    b = pl.program_id(0)

    @pl.when(b == 0)
    def _softmax():
        adj = adj_ref[...]
        m = jnp.max(adj, axis=-1, keepdims=True)
        e = jnp.exp(adj - m)
        a_ref[...] = e / jnp.sum(e, axis=-1, keepdims=True)

    def dot(p, q):
        return jax.lax.dot(p, q, preferred_element_type=jnp.float32)

    x = x_ref[0]
    a = a_ref[...]
    wf = wf_ref[...]
    bf = bf_ref[...]
    wg2 = wg2_ref[...]

    base = dot(x, wg1_ref[...]) + bg_ref[...]
    u1 = jnp.maximum(dot(dot(a, x), wf) + bf, 0.0)
    f1 = base + dot(u1, wg2)
    u2 = jnp.maximum(dot(dot(a, f1), wf) + bf, 0.0)
    o_ref[0] = base + dot(u2, wg2)


def kernel(features, adj, W_fub, b_fub, W_g, b_g):
    wg1 = W_g[:_C]
    wg2 = W_g[_C:]
    bf = b_fub.reshape(1, _C)
    bg = b_g.reshape(1, _C)
    return pl.pallas_call(
        _gcn_fpn_body,
        grid=(_B,),
        in_specs=[
            pl.BlockSpec((1, _N, _C), lambda b: (b, 0, 0)),
            pl.BlockSpec((_N, _N), lambda b: (0, 0)),
            pl.BlockSpec((_C, _C), lambda b: (0, 0)),
            pl.BlockSpec((1, _C), lambda b: (0, 0)),
            pl.BlockSpec((_C, _C), lambda b: (0, 0)),
            pl.BlockSpec((_C, _C), lambda b: (0, 0)),
            pl.BlockSpec((1, _C), lambda b: (0, 0)),
        ],
        out_specs=pl.BlockSpec((1, _N, _C), lambda b: (b, 0, 0)),
        out_shape=jax.ShapeDtypeStruct((_B, _N, _C), jnp.float32),
        scratch_shapes=[pltpu.VMEM((_N, _N), jnp.float32)],
    )(features, adj, W_fub, bf, wg1, wg2, bg)
